# baseline (speedup 1.0000x reference)
import functools

import jax
import jax.numpy as jnp
from jax import lax
from jax.experimental import pallas as pl
from jax.experimental.pallas import tpu as pltpu
from jax.experimental.pallas import tpu_sc as plsc
from jax.experimental.layout import Layout, with_layout_constraint

L = 16


def _make_sc_kernel(B, S, D, SB):
    info = plsc.get_sparse_core_info()
    NC, NS = info.num_cores, info.num_subcores
    NW = NC * NS
    assert B % (NW * 128) == 0 and B // NW == 128
    assert S % (2 * SB) == 0
    G = S // SB
    DR = D // 8
    TILE = 8 * 128
    SROW = DR * 32 * TILE
    NJ = B // 128
    ROWS = SB * 128

    mesh = plsc.VectorSubcoreMesh(core_axis_name="c", subcore_axis_name="s")

    @functools.partial(
        pl.kernel,
        mesh=mesh,
        out_type=jax.ShapeDtypeStruct((B * S * D,), jnp.float32),
        compiler_params=pltpu.CompilerParams(
            use_tc_tiling_on_sc=False, needs_layout_passes=False),
        scratch_types=[
            pltpu.VMEM((ROWS,), jnp.int32),
            pltpu.VMEM((ROWS,), jnp.int32),
            pltpu.VMEM((ROWS, D), jnp.float32),
            pltpu.VMEM((ROWS, D), jnp.float32),
            pltpu.VMEM((SB * DR * TILE,), jnp.float32),
            pltpu.VMEM((SB * DR * TILE,), jnp.float32),
            pltpu.VMEM((S, D), jnp.float32),
            pltpu.SemaphoreType.DMA,
            pltpu.SemaphoreType.DMA,
            pltpu.SemaphoreType.DMA,
            pltpu.SemaphoreType.DMA,
            pltpu.SemaphoreType.DMA,
            pltpu.SemaphoreType.DMA,
        ],
    )
    def emb_kernel(xt_ref, tab_ref, pos_ref, out_ref,
                   idx_a, idx_b, in_a, in_b, stg_a, stg_b, pos_v,
                   sia, sib, sga, sgb, swa, swb):
        wid = lax.axis_index("s") * NC + lax.axis_index("c")
        bcol0 = wid * 128
        pltpu.sync_copy(pos_ref, pos_v)

        lane = lax.iota(jnp.int32, L)
        offs = []
        for j in range(D // L):
            d = lane + (16 * j)
            offs.append(
                lax.shift_left(lax.shift_right_logical(d, 3), 10)
                + lax.shift_left(lax.bitwise_and(d, 7), 7))

        def fire_idx(g, idxb, sem):
            cps = []
            for si in range(SB):
                s = g * SB + si
                cps.append(pltpu.async_copy(
                    xt_ref.at[pl.ds(s * B + bcol0, 128)],
                    idxb.at[pl.ds(si * 128, 128)], sem))
            return cps

        def wait_idx(idxb, sem):
            for si in range(SB):
                pltpu.make_async_copy(
                    xt_ref.at[pl.ds(si * 128, 128)],
                    idxb.at[pl.ds(si * 128, 128)], sem).wait()

        def fire_gather(idxb, inb, sem):
            for si in range(SB):
                pltpu.async_copy(
                    tab_ref.at[idxb.at[pl.ds(si * 128, 128)]],
                    inb.at[pl.ds(si * 128, 128)], sem)

        def wait_gather(idxb, inb, sem):
            for si in range(SB):
                pltpu.make_async_copy(
                    tab_ref.at[idxb.at[pl.ds(si * 128, 128)]],
                    inb.at[pl.ds(si * 128, 128)], sem).wait()

        def add_transpose(g, inb, stgb):
            for si in range(SB):
                s = g * SB + si
                prow = [pos_v[s, pl.ds(16 * j, L)] for j in range(D // L)]
                sbase = si * DR * TILE

                def brow_body(c, carry):
                    r = si * 128 + c
                    base = jnp.full((L,), sbase + c, dtype=jnp.int32)
                    for j in range(D // L):
                        v = inb[r, pl.ds(16 * j, L)] + prow[j]
                        plsc.store_scatter(stgb, [offs[j] + base], v)
                    return carry

                lax.fori_loop(0, 128, brow_body, 0)

        def fire_out(g, stgb, sem):
            for si in range(SB):
                s = g * SB + si
                for dr in range(DR):
                    pltpu.async_copy(
                        stgb.at[pl.ds((si * DR + dr) * TILE, TILE)],
                        out_ref.at[pl.ds(
                            s * SROW + dr * NJ * TILE + wid * 1024, TILE)],
                        sem)

        def wait_out(stgb, sem):
            for si in range(SB):
                for dr in range(DR):
                    pltpu.make_async_copy(
                        stgb.at[pl.ds((si * DR + dr) * TILE, TILE)],
                        out_ref.at[pl.ds((si * DR + dr) * TILE, TILE)],
                        sem).wait()

        fire_idx(0, idx_a, sia)
        wait_idx(idx_a, sia)
        fire_gather(idx_a, in_a, sga)
        fire_idx(1, idx_b, sib)

        def body(h, carry):
            ga = 2 * h
            gb = 2 * h + 1
            wait_idx(idx_b, sib)
            fire_gather(idx_b, in_b, sgb)
            wait_gather(idx_a, in_a, sga)

            @pl.when(h > 0)
            def _():
                wait_out(stg_a, swa)
            add_transpose(ga, in_a, stg_a)
            fire_out(ga, stg_a, swa)

            @pl.when(h + 1 < G // 2)
            def _():
                fire_idx(ga + 2, idx_a, sia)
                wait_idx(idx_a, sia)
                fire_gather(idx_a, in_a, sga)
            wait_gather(idx_b, in_b, sgb)

            @pl.when(h > 0)
            def _():
                wait_out(stg_b, swb)
            add_transpose(gb, in_b, stg_b)
            fire_out(gb, stg_b, swb)

            @pl.when(h + 1 < G // 2)
            def _():
                fire_idx(gb + 2, idx_b, sib)
            return carry

        lax.fori_loop(0, G // 2, body, 0)
        wait_out(stg_a, swa)
        wait_out(stg_b, swb)

    return emb_kernel


def kernel(x, word_table, pos_table):
    B, S = x.shape
    V, D = word_table.shape
    xt_flat = x.T.reshape(-1).astype(jnp.int32)
    emb_fn = _make_sc_kernel(B, S, D, SB=2)
    flat = emb_fn(xt_flat, word_table, pos_table)
    v = flat.reshape(S, D // 8, B // 128, 8, 128)
    v = v.transpose(0, 1, 3, 2, 4)
    v = v.reshape(S, D, B)
    v = v.transpose(2, 0, 1)
    return with_layout_constraint(
        v, Layout(major_to_minor=(1, 2, 0), tiling=((8, 128),)))


# speedup vs baseline: 1.0015x; 1.0015x over previous
import functools

import jax
import jax.numpy as jnp
from jax import lax
from jax.experimental import pallas as pl
from jax.experimental.pallas import tpu as pltpu
from jax.experimental.pallas import tpu_sc as plsc
from jax.experimental.layout import Layout, with_layout_constraint

L = 16


def _make_sc_kernel(B, S, D, SB):
    info = plsc.get_sparse_core_info()
    NC, NS = info.num_cores, info.num_subcores
    NW = NC * NS
    assert B % (NW * 128) == 0 and B // NW == 128
    assert S % (2 * SB) == 0
    G = S // SB
    DR = D // 8
    TILE = 8 * 128
    SROW = DR * 32 * TILE
    NJ = B // 128
    ROWS = SB * 128

    mesh = plsc.VectorSubcoreMesh(core_axis_name="c", subcore_axis_name="s")

    @functools.partial(
        pl.kernel,
        mesh=mesh,
        out_type=jax.ShapeDtypeStruct((B * S * D,), jnp.float32),
        compiler_params=pltpu.CompilerParams(
            use_tc_tiling_on_sc=False, needs_layout_passes=False,
            disable_bounds_checks=True),
        scratch_types=[
            pltpu.VMEM((ROWS,), jnp.int32),
            pltpu.VMEM((ROWS,), jnp.int32),
            pltpu.VMEM((ROWS, D), jnp.float32),
            pltpu.VMEM((ROWS, D), jnp.float32),
            pltpu.VMEM((SB * DR * TILE,), jnp.float32),
            pltpu.VMEM((SB * DR * TILE,), jnp.float32),
            pltpu.VMEM((S, D), jnp.float32),
            pltpu.SemaphoreType.DMA,
            pltpu.SemaphoreType.DMA,
            pltpu.SemaphoreType.DMA,
            pltpu.SemaphoreType.DMA,
            pltpu.SemaphoreType.DMA,
            pltpu.SemaphoreType.DMA,
        ],
    )
    def emb_kernel(xt_ref, tab_ref, pos_ref, out_ref,
                   idx_a, idx_b, in_a, in_b, stg_a, stg_b, pos_v,
                   sia, sib, sga, sgb, swa, swb):
        wid = lax.axis_index("s") * NC + lax.axis_index("c")
        bcol0 = wid * 128
        pltpu.sync_copy(pos_ref, pos_v)

        lane = lax.iota(jnp.int32, L)
        offs = []
        for j in range(D // L):
            d = lane + (16 * j)
            offs.append(
                lax.shift_left(lax.shift_right_logical(d, 3), 10)
                + lax.shift_left(lax.bitwise_and(d, 7), 7))

        def fire_idx(g, idxb, sem):
            cps = []
            for si in range(SB):
                s = g * SB + si
                cps.append(pltpu.async_copy(
                    xt_ref.at[pl.ds(s * B + bcol0, 128)],
                    idxb.at[pl.ds(si * 128, 128)], sem))
            return cps

        def wait_idx(idxb, sem):
            for si in range(SB):
                pltpu.make_async_copy(
                    xt_ref.at[pl.ds(si * 128, 128)],
                    idxb.at[pl.ds(si * 128, 128)], sem).wait()

        def fire_gather(idxb, inb, sem):
            for si in range(SB):
                pltpu.async_copy(
                    tab_ref.at[idxb.at[pl.ds(si * 128, 128)]],
                    inb.at[pl.ds(si * 128, 128)], sem)

        def wait_gather(idxb, inb, sem):
            for si in range(SB):
                pltpu.make_async_copy(
                    tab_ref.at[idxb.at[pl.ds(si * 128, 128)]],
                    inb.at[pl.ds(si * 128, 128)], sem).wait()

        def add_transpose(g, inb, stgb):
            for si in range(SB):
                s = g * SB + si
                prow = [pos_v[s, pl.ds(16 * j, L)] for j in range(D // L)]
                sbase = si * DR * TILE

                def brow_body(c, carry):
                    r = si * 128 + c
                    base = jnp.full((L,), sbase + c, dtype=jnp.int32)
                    for j in range(D // L):
                        v = inb[r, pl.ds(16 * j, L)] + prow[j]
                        plsc.store_scatter(stgb, [offs[j] + base], v)
                    return carry

                lax.fori_loop(0, 128, brow_body, 0)

        def fire_out(g, stgb, sem):
            for si in range(SB):
                s = g * SB + si
                for dr in range(DR):
                    pltpu.async_copy(
                        stgb.at[pl.ds((si * DR + dr) * TILE, TILE)],
                        out_ref.at[pl.ds(
                            s * SROW + dr * NJ * TILE + wid * 1024, TILE)],
                        sem)

        def wait_out(stgb, sem):
            for si in range(SB):
                for dr in range(DR):
                    pltpu.make_async_copy(
                        stgb.at[pl.ds((si * DR + dr) * TILE, TILE)],
                        out_ref.at[pl.ds((si * DR + dr) * TILE, TILE)],
                        sem).wait()

        fire_idx(0, idx_a, sia)
        wait_idx(idx_a, sia)
        fire_gather(idx_a, in_a, sga)
        fire_idx(1, idx_b, sib)

        def body(h, carry):
            ga = 2 * h
            gb = 2 * h + 1
            wait_idx(idx_b, sib)
            fire_gather(idx_b, in_b, sgb)
            wait_gather(idx_a, in_a, sga)

            @pl.when(h > 0)
            def _():
                wait_out(stg_a, swa)
            add_transpose(ga, in_a, stg_a)
            fire_out(ga, stg_a, swa)

            @pl.when(h + 1 < G // 2)
            def _():
                fire_idx(ga + 2, idx_a, sia)
                wait_idx(idx_a, sia)
                fire_gather(idx_a, in_a, sga)
            wait_gather(idx_b, in_b, sgb)

            @pl.when(h > 0)
            def _():
                wait_out(stg_b, swb)
            add_transpose(gb, in_b, stg_b)
            fire_out(gb, stg_b, swb)

            @pl.when(h + 1 < G // 2)
            def _():
                fire_idx(gb + 2, idx_b, sib)
            return carry

        lax.fori_loop(0, G // 2, body, 0)
        wait_out(stg_a, swa)
        wait_out(stg_b, swb)

    return emb_kernel


def kernel(x, word_table, pos_table):
    B, S = x.shape
    V, D = word_table.shape
    xt_flat = x.T.reshape(-1).astype(jnp.int32)
    emb_fn = _make_sc_kernel(B, S, D, SB=2)
    flat = emb_fn(xt_flat, word_table, pos_table)
    v = flat.reshape(S, D // 8, B // 128, 8, 128)
    v = v.transpose(0, 1, 3, 2, 4)
    v = v.reshape(S, D, B)
    v = v.transpose(2, 0, 1)
    return with_layout_constraint(
        v, Layout(major_to_minor=(1, 2, 0), tiling=((8, 128),)))


# speedup vs baseline: 1.0121x; 1.0105x over previous
import functools

import jax
import jax.numpy as jnp
from jax import lax
from jax.experimental import pallas as pl
from jax.experimental.pallas import tpu as pltpu
from jax.experimental.pallas import tpu_sc as plsc
from jax.experimental.layout import Layout, with_layout_constraint

L = 16


def _make_sc_kernel(B, S, D, SB):
    info = plsc.get_sparse_core_info()
    NC, NS = info.num_cores, info.num_subcores
    NW = NC * NS
    assert B % (NW * 128) == 0 and B // NW == 128
    assert S % (2 * SB) == 0
    G = S // SB
    DR = D // 8
    TILE = 8 * 128
    SROW = DR * 32 * TILE
    NJ = B // 128
    ROWS = SB * 128

    mesh = plsc.VectorSubcoreMesh(core_axis_name="c", subcore_axis_name="s")

    @functools.partial(
        pl.kernel,
        mesh=mesh,
        out_type=jax.ShapeDtypeStruct((B * S * D,), jnp.float32),
        compiler_params=pltpu.CompilerParams(
            use_tc_tiling_on_sc=False, needs_layout_passes=False,
            disable_bounds_checks=True),
        scratch_types=[
            pltpu.VMEM((ROWS,), jnp.int32),
            pltpu.VMEM((ROWS,), jnp.int32),
            pltpu.VMEM((ROWS, D), jnp.float32),
            pltpu.VMEM((ROWS, D), jnp.float32),
            pltpu.VMEM((SB * DR * TILE,), jnp.float32),
            pltpu.VMEM((SB * DR * TILE,), jnp.float32),
            pltpu.VMEM((S, D), jnp.float32),
            pltpu.SemaphoreType.DMA,
            pltpu.SemaphoreType.DMA,
            pltpu.SemaphoreType.DMA,
            pltpu.SemaphoreType.DMA,
            pltpu.SemaphoreType.DMA,
            pltpu.SemaphoreType.DMA,
        ],
    )
    def emb_kernel(xt_ref, tab_ref, pos_ref, out_ref,
                   idx_a, idx_b, in_a, in_b, stg_a, stg_b, pos_v,
                   sia, sib, sga, sgb, swa, swb):
        wid = lax.axis_index("s") * NC + lax.axis_index("c")
        bcol0 = wid * 128
        pltpu.sync_copy(pos_ref, pos_v)

        lane = lax.iota(jnp.int32, L)
        offs = []
        for j in range(D // L):
            d = lane + (16 * j)
            offs.append(
                lax.shift_left(lax.shift_right_logical(d, 3), 10)
                + lax.shift_left(lax.bitwise_and(d, 7), 7))

        def fire_idx(g, idxb, sem):
            cps = []
            for si in range(SB):
                s = g * SB + si
                cps.append(pltpu.async_copy(
                    xt_ref.at[pl.ds(s * B + bcol0, 128)],
                    idxb.at[pl.ds(si * 128, 128)], sem))
            return cps

        def wait_idx(idxb, sem):
            for si in range(SB):
                pltpu.make_async_copy(
                    xt_ref.at[pl.ds(si * 128, 128)],
                    idxb.at[pl.ds(si * 128, 128)], sem).wait()

        def fire_gather(idxb, inb, sem):
            for si in range(SB):
                pltpu.async_copy(
                    tab_ref.at[idxb.at[pl.ds(si * 128, 128)]],
                    inb.at[pl.ds(si * 128, 128)], sem)

        def wait_gather(idxb, inb, sem):
            for si in range(SB):
                pltpu.make_async_copy(
                    tab_ref.at[idxb.at[pl.ds(si * 128, 128)]],
                    inb.at[pl.ds(si * 128, 128)], sem).wait()

        def add_transpose(g, inb, stgb):
            for si in range(SB):
                s = g * SB + si
                prow = [pos_v[s, pl.ds(16 * j, L)] for j in range(D // L)]
                sbase = si * DR * TILE

                def brow_body(c4, carry):
                    c0 = c4 * 4
                    for u in range(4):
                        c = c0 + u
                        r = si * 128 + c
                        base = jnp.full((L,), sbase + c, dtype=jnp.int32)
                        for j in range(D // L):
                            v = inb[r, pl.ds(16 * j, L)] + prow[j]
                            plsc.store_scatter(stgb, [offs[j] + base], v)
                    return carry

                lax.fori_loop(0, 32, brow_body, 0)

        def fire_out(g, stgb, sem):
            for si in range(SB):
                s = g * SB + si
                for dr in range(DR):
                    pltpu.async_copy(
                        stgb.at[pl.ds((si * DR + dr) * TILE, TILE)],
                        out_ref.at[pl.ds(
                            s * SROW + dr * NJ * TILE + wid * 1024, TILE)],
                        sem)

        def wait_out(stgb, sem):
            for si in range(SB):
                for dr in range(DR):
                    pltpu.make_async_copy(
                        stgb.at[pl.ds((si * DR + dr) * TILE, TILE)],
                        out_ref.at[pl.ds((si * DR + dr) * TILE, TILE)],
                        sem).wait()

        fire_idx(0, idx_a, sia)
        wait_idx(idx_a, sia)
        fire_gather(idx_a, in_a, sga)
        fire_idx(1, idx_b, sib)

        def body(h, carry):
            ga = 2 * h
            gb = 2 * h + 1
            wait_idx(idx_b, sib)
            fire_gather(idx_b, in_b, sgb)
            wait_gather(idx_a, in_a, sga)

            @pl.when(h > 0)
            def _():
                wait_out(stg_a, swa)
            add_transpose(ga, in_a, stg_a)
            fire_out(ga, stg_a, swa)

            @pl.when(h + 1 < G // 2)
            def _():
                fire_idx(ga + 2, idx_a, sia)
                wait_idx(idx_a, sia)
                fire_gather(idx_a, in_a, sga)
            wait_gather(idx_b, in_b, sgb)

            @pl.when(h > 0)
            def _():
                wait_out(stg_b, swb)
            add_transpose(gb, in_b, stg_b)
            fire_out(gb, stg_b, swb)

            @pl.when(h + 1 < G // 2)
            def _():
                fire_idx(gb + 2, idx_b, sib)
            return carry

        lax.fori_loop(0, G // 2, body, 0)
        wait_out(stg_a, swa)
        wait_out(stg_b, swb)

    return emb_kernel


def kernel(x, word_table, pos_table):
    B, S = x.shape
    V, D = word_table.shape
    xt_flat = x.T.reshape(-1).astype(jnp.int32)
    emb_fn = _make_sc_kernel(B, S, D, SB=2)
    flat = emb_fn(xt_flat, word_table, pos_table)
    v = flat.reshape(S, D // 8, B // 128, 8, 128)
    v = v.transpose(0, 1, 3, 2, 4)
    v = v.reshape(S, D, B)
    v = v.transpose(2, 0, 1)
    return with_layout_constraint(
        v, Layout(major_to_minor=(1, 2, 0), tiling=((8, 128),)))


# speedup vs baseline: 1.0460x; 1.0336x over previous
import functools

import jax
import jax.numpy as jnp
from jax import lax
from jax.experimental import pallas as pl
from jax.experimental.pallas import tpu as pltpu
from jax.experimental.pallas import tpu_sc as plsc
from jax.experimental.layout import Layout, with_layout_constraint

L = 16


def _make_sc_kernel(B, S, D, SB):
    info = plsc.get_sparse_core_info()
    NC, NS = info.num_cores, info.num_subcores
    NW = NC * NS
    assert B % (NW * 128) == 0 and B // NW == 128
    assert S % (2 * SB) == 0
    G = S // SB
    DR = D // 8
    TILE = 8 * 128
    SROW = DR * 32 * TILE
    NJ = B // 128
    ROWS = SB * 128

    mesh = plsc.VectorSubcoreMesh(core_axis_name="c", subcore_axis_name="s")

    @functools.partial(
        pl.kernel,
        mesh=mesh,
        out_type=jax.ShapeDtypeStruct((B * S * D,), jnp.float32),
        compiler_params=pltpu.CompilerParams(
            use_tc_tiling_on_sc=False, needs_layout_passes=False,
            disable_bounds_checks=True),
        scratch_types=[
            pltpu.VMEM((S, 128), jnp.int32),
            pltpu.VMEM((ROWS, D), jnp.float32),
            pltpu.VMEM((ROWS, D), jnp.float32),
            pltpu.VMEM((SB * DR * TILE,), jnp.float32),
            pltpu.VMEM((SB * DR * TILE,), jnp.float32),
            pltpu.VMEM((S, D), jnp.float32),
            pltpu.SemaphoreType.DMA,
            pltpu.SemaphoreType.DMA,
            pltpu.SemaphoreType.DMA,
            pltpu.SemaphoreType.DMA,
        ],
    )
    def emb_kernel(xt_ref, tab_ref, pos_ref, out_ref,
                   idx_all, in_a, in_b, stg_a, stg_b, pos_v,
                   sga, sgb, swa, swb):
        wid = lax.axis_index("s") * NC + lax.axis_index("c")
        bcol0 = wid * 128
        pltpu.sync_copy(pos_ref, pos_v)
        pltpu.sync_copy(xt_ref.at[:, pl.ds(bcol0, 128)], idx_all)

        lane = lax.iota(jnp.int32, L)
        offs = []
        for j in range(D // L):
            d = lane + (16 * j)
            offs.append(
                lax.shift_left(lax.shift_right_logical(d, 3), 10)
                + lax.shift_left(lax.bitwise_and(d, 7), 7))

        def fire_gather(g, inb, sem):
            for si in range(SB):
                pltpu.async_copy(
                    tab_ref.at[idx_all.at[g * SB + si]],
                    inb.at[pl.ds(si * 128, 128)], sem)

        def wait_gather(inb, sem):
            for si in range(SB):
                pltpu.make_async_copy(
                    tab_ref.at[idx_all.at[si]],
                    inb.at[pl.ds(si * 128, 128)], sem).wait()

        def add_transpose(g, inb, stgb):
            for si in range(SB):
                s = g * SB + si
                prow = [pos_v[s, pl.ds(16 * j, L)] for j in range(D // L)]
                sbase = si * DR * TILE

                def brow_body(c4, carry):
                    c0 = c4 * 4
                    for u in range(4):
                        c = c0 + u
                        r = si * 128 + c
                        base = jnp.full((L,), sbase + c, dtype=jnp.int32)
                        for j in range(D // L):
                            v = inb[r, pl.ds(16 * j, L)] + prow[j]
                            plsc.store_scatter(stgb, [offs[j] + base], v)
                    return carry

                lax.fori_loop(0, 32, brow_body, 0)

        def fire_out(g, stgb, sem):
            for si in range(SB):
                s = g * SB + si
                for dr in range(DR):
                    pltpu.async_copy(
                        stgb.at[pl.ds((si * DR + dr) * TILE, TILE)],
                        out_ref.at[pl.ds(
                            s * SROW + dr * NJ * TILE + wid * 1024, TILE)],
                        sem)

        def wait_out(stgb, sem):
            for si in range(SB):
                for dr in range(DR):
                    pltpu.make_async_copy(
                        stgb.at[pl.ds((si * DR + dr) * TILE, TILE)],
                        out_ref.at[pl.ds((si * DR + dr) * TILE, TILE)],
                        sem).wait()

        fire_gather(0, in_a, sga)
        fire_gather(1, in_b, sgb)

        def body(h, carry):
            ga = 2 * h
            gb = 2 * h + 1
            wait_gather(in_a, sga)

            @pl.when(h > 0)
            def _():
                wait_out(stg_a, swa)
            add_transpose(ga, in_a, stg_a)
            fire_out(ga, stg_a, swa)

            @pl.when(h + 1 < G // 2)
            def _():
                fire_gather(ga + 2, in_a, sga)
            wait_gather(in_b, sgb)

            @pl.when(h > 0)
            def _():
                wait_out(stg_b, swb)
            add_transpose(gb, in_b, stg_b)
            fire_out(gb, stg_b, swb)

            @pl.when(h + 1 < G // 2)
            def _():
                fire_gather(gb + 2, in_b, sgb)
            return carry

        lax.fori_loop(0, G // 2, body, 0)
        wait_out(stg_a, swa)
        wait_out(stg_b, swb)

    return emb_kernel


def kernel(x, word_table, pos_table):
    B, S = x.shape
    V, D = word_table.shape
    xt = x.T.astype(jnp.int32)
    emb_fn = _make_sc_kernel(B, S, D, SB=2)
    flat = emb_fn(xt, word_table, pos_table)
    v = flat.reshape(S, D // 8, B // 128, 8, 128)
    v = v.transpose(0, 1, 3, 2, 4)
    v = v.reshape(S, D, B)
    v = v.transpose(2, 0, 1)
    return with_layout_constraint(
        v, Layout(major_to_minor=(1, 2, 0), tiling=((8, 128),)))


# speedup vs baseline: 1.8977x; 1.8141x over previous
import functools

import jax
import jax.numpy as jnp
from jax import lax
from jax.experimental import pallas as pl
from jax.experimental.pallas import tpu as pltpu
from jax.experimental.pallas import tpu_sc as plsc
from jax.experimental.layout import Layout, with_layout_constraint

L = 16


def _make_sc_kernel(B, S, D, NB):
    info = plsc.get_sparse_core_info()
    NC, NS = info.num_cores, info.num_subcores
    NW = NC * NS
    assert B % NW == 0
    BPW = B // NW
    assert BPW % NB == 0
    ROWS = NB * S
    CH = 80
    assert ROWS % CH == 0
    NCH = ROWS // CH
    G = BPW // NB
    assert G % 2 == 0
    WROWS = BPW * S

    mesh = plsc.VectorSubcoreMesh(core_axis_name="c", subcore_axis_name="s")

    @functools.partial(
        pl.kernel,
        mesh=mesh,
        out_type=jax.ShapeDtypeStruct((B * S, 2 * D), jnp.float32),
        compiler_params=pltpu.CompilerParams(use_tc_tiling_on_sc=False),
        scratch_types=[
            pltpu.VMEM((WROWS,), jnp.int32),
            pltpu.VMEM((ROWS, D), jnp.float32),
            pltpu.VMEM((ROWS, D), jnp.float32),
            pltpu.VMEM((S, D), jnp.float32),
            pltpu.SemaphoreType.DMA,
            pltpu.SemaphoreType.DMA,
            pltpu.SemaphoreType.DMA,
            pltpu.SemaphoreType.DMA,
        ],
    )
    def emb_kernel(x_ref, tab_ref, pos_ref, out_ref,
                   idx_v, emb_a, emb_b, pos_v, sga, sgb, swa, swb):
        wid = lax.axis_index("s") * NC + lax.axis_index("c")
        row_base = wid * WROWS
        pltpu.sync_copy(x_ref.at[pl.ds(row_base, WROWS)], idx_v)
        pltpu.sync_copy(pos_ref, pos_v)

        def fire_gather(g, buf, sem):
            for i in range(NCH):
                pltpu.async_copy(
                    tab_ref.at[idx_v.at[pl.ds(g * ROWS + i * CH, CH)]],
                    buf.at[pl.ds(i * CH, CH)],
                    sem,
                )

        def wait_gather(buf, sem):
            for i in range(NCH):
                pltpu.make_async_copy(
                    tab_ref.at[idx_v.at[pl.ds(i * CH, CH)]],
                    buf.at[pl.ds(i * CH, CH)],
                    sem,
                ).wait()

        def fire_writeout(g, buf, sem):
            return pltpu.async_copy(
                buf,
                out_ref.at[pl.ds(row_base + g * ROWS, ROWS), pl.ds(0, D)],
                sem)

        def wait_writeout(buf, sem):
            pltpu.make_async_copy(
                buf,
                out_ref.at[pl.ds(row_base, ROWS), pl.ds(0, D)],
                sem).wait()

        def add_pos(buf):
            def add_s(s, c2):
                for j in range(NB):
                    r = j * S + s
                    for cchunk in range(D // L):
                        sl = pl.ds(cchunk * L, L)
                        buf[r, sl] = buf[r, sl] + pos_v[s, sl]
                return c2
            lax.fori_loop(0, S, add_s, 0)

        fire_gather(0, emb_a, sga)

        def body(h, carry):
            ga = 2 * h
            gb = 2 * h + 1
            @pl.when(h > 0)
            def _():
                wait_writeout(emb_b, swb)
            fire_gather(gb, emb_b, sgb)
            wait_gather(emb_a, sga)
            add_pos(emb_a)
            fire_writeout(ga, emb_a, swa)
            @pl.when(h + 1 < G // 2)
            def _():
                wait_writeout(emb_a, swa)
                fire_gather(ga + 2, emb_a, sga)
            wait_gather(emb_b, sgb)
            add_pos(emb_b)
            fire_writeout(gb, emb_b, swb)
            return carry

        lax.fori_loop(0, G // 2, body, 0)
        wait_writeout(emb_a, swa)
        wait_writeout(emb_b, swb)

    return emb_kernel


def kernel(x, word_table, pos_table):
    B, S = x.shape
    V, D = word_table.shape
    x_flat = x.reshape(-1).astype(jnp.int32)
    emb_fn = _make_sc_kernel(B, S, D, NB=2)
    out_pad = emb_fn(x_flat, word_table, pos_table)
    out3 = out_pad[:, :D].reshape(B, S, D)
    return with_layout_constraint(
        out3, Layout(major_to_minor=(1, 2, 0), tiling=((8, 128),)))
